# baseline (speedup 1.0000x reference)
import jax
import jax.numpy as jnp
import numpy as np
from jax.experimental import pallas as pl

_P = 2
_C = 96
_IMG = 512
_W = _IMG // _P
_H = _IMG // _P
_N = _H * _W
_R = 128
_NSTEPS = _H // _R


def _dinv_grid() -> np.ndarray:
    r = np.arange(_H)[:, None]
    c = np.arange(_W)[None, :]
    deg = (1.0 + (r > 0) + (r < _H - 1) + (c > 0) + (c < _W - 1)
           + ((r == _H - 2) & (c == _W - 2)))
    return (1.0 / np.sqrt(deg)).astype(np.float32)


def _build_perm() -> np.ndarray:
    s = np.zeros((2 * _IMG, 4 * _W), dtype=np.float32)
    for kh in range(_P):
        for kw in range(_P):
            k = _P * kh + kw
            for c in range(_W):
                s[_IMG * kh + _P * c + kw, k * _W + c] = 1.0
    return s


def _build_dsrc() -> np.ndarray:
    dinv = _dinv_grid()
    out = np.zeros((_NSTEPS, _R + 2, 4 * _W), dtype=np.float32)
    for i in range(_NSTEPS):
        for rl in range(_R + 2):
            r = i * _R + rl - 1
            if 0 <= r < _H:
                out[i, rl, :] = np.tile(dinv[r], 4)
    return out


def _build_lane_masks():
    ml = np.ones((1, 4 * _W), dtype=np.float32)
    mr = np.ones((1, 4 * _W), dtype=np.float32)
    for k in range(4):
        ml[0, k * _W] = 0.0
        mr[0, (k + 1) * _W - 1] = 0.0
    return ml, mr


_SM = _build_perm()
_DSRC = _build_dsrc()
_ML, _MR = _build_lane_masks()
_DDST = _dinv_grid().reshape(_NSTEPS, 1, _R * _W)


def _body(xpv_ref, xcv_ref, xnv_ref, sm_ref, wf_ref, wg_ref, b_ref, dv_ref,
          ml_ref, mr_ref, dd_ref, out_ref):
    i = pl.program_id(0)
    wct = jnp.dot(wg_ref[...], wf_ref[...], preferred_element_type=jnp.float32)

    xb = jnp.concatenate(
        [xpv_ref[2 * _R - 2:, :], xcv_ref[...], xnv_ref[:2, :]],
        axis=0)
    xh = jnp.concatenate(
        [jnp.concatenate([xb[2 * rl:2 * rl + 1, :],
                          xb[2 * rl + 1:2 * rl + 2, :]], axis=1)
         for rl in range(_R + 2)], axis=0)

    ut = jnp.dot(xh, sm_ref[...], preferred_element_type=jnp.float32)
    ut = ut * dv_ref[0]

    ct = ut[1:_R + 1]
    zc1 = jnp.zeros((_R, 1), jnp.float32)
    shl = jnp.concatenate([zc1, ct[:, :4 * _W - 1]], axis=1) * ml_ref[...]
    shr = jnp.concatenate([ct[:, 1:], zc1], axis=1) * mr_ref[...]
    usum = ut[0:_R] + ct + ut[2:_R + 2] + shl + shr

    pt = jnp.concatenate(
        [jnp.concatenate([usum[r:r + 1, k * _W:(k + 1) * _W]
                          for r in range(_R)], axis=1)
         for k in range(4)], axis=0)
    z = jnp.dot(wct, pt, preferred_element_type=jnp.float32)
    out_ref[...] = z * dd_ref[0] + b_ref[...]

    @pl.when(i == _NSTEPS - 1)
    def _():
        loc = ((_H - 2) % _R) * _W + (_W - 2)
        src = jnp.concatenate(
            [ut[_R:_R + 1, (k + 1) * _W - 1:(k + 1) * _W]
             for k in range(4)], axis=0)
        corr = np.float32(1.0 / np.sqrt(6.0)) * jnp.dot(
            wct, src, preferred_element_type=jnp.float32)
        out_ref[:, loc:loc + 1] = out_ref[:, loc:loc + 1] + corr


def kernel(x, Wconv, Wgcn, bgcn):
    xr = x.reshape(_IMG, _IMG)
    wf = Wconv.reshape(_C, 4).astype(jnp.float32)
    wg = Wgcn.astype(jnp.float32)
    b = bgcn.reshape(_C, 1).astype(jnp.float32)

    out = pl.pallas_call(
        _body,
        grid=(_NSTEPS,),
        in_specs=[
            pl.BlockSpec((2 * _R, _IMG), lambda i: (jnp.maximum(i - 1, 0), 0)),
            pl.BlockSpec((2 * _R, _IMG), lambda i: (i, 0)),
            pl.BlockSpec((2 * _R, _IMG),
                         lambda i: (jnp.minimum(i + 1, _NSTEPS - 1), 0)),
            pl.BlockSpec((2 * _IMG, 4 * _W), lambda i: (0, 0)),
            pl.BlockSpec((_C, 4), lambda i: (0, 0)),
            pl.BlockSpec((_C, _C), lambda i: (0, 0)),
            pl.BlockSpec((_C, 1), lambda i: (0, 0)),
            pl.BlockSpec((1, _R + 2, 4 * _W), lambda i: (i, 0, 0)),
            pl.BlockSpec((1, 4 * _W), lambda i: (0, 0)),
            pl.BlockSpec((1, 4 * _W), lambda i: (0, 0)),
            pl.BlockSpec((1, 1, _R * _W), lambda i: (i, 0, 0)),
        ],
        out_specs=pl.BlockSpec((_C, _R * _W), lambda i: (0, i)),
        out_shape=jax.ShapeDtypeStruct((_C, _N), jnp.float32),
    )(xr, xr, xr, jnp.asarray(_SM), wf, wg, b, jnp.asarray(_DSRC),
      jnp.asarray(_ML), jnp.asarray(_MR), jnp.asarray(_DDST))

    return out.T.reshape(1, _N, _C)


# speedup vs baseline: 1.0380x; 1.0380x over previous
import jax
import jax.numpy as jnp
import numpy as np
from jax.experimental import pallas as pl

_P = 2
_C = 96
_IMG = 512
_W = _IMG // _P
_H = _IMG // _P
_N = _H * _W
_R = 64
_NSTEPS = _H // _R


def _dinv_grid() -> np.ndarray:
    r = np.arange(_H)[:, None]
    c = np.arange(_W)[None, :]
    deg = (1.0 + (r > 0) + (r < _H - 1) + (c > 0) + (c < _W - 1)
           + ((r == _H - 2) & (c == _W - 2)))
    return (1.0 / np.sqrt(deg)).astype(np.float32)


def _build_perm() -> np.ndarray:
    s = np.zeros((2 * _IMG, 4 * _W), dtype=np.float32)
    for kh in range(_P):
        for kw in range(_P):
            k = _P * kh + kw
            for c in range(_W):
                s[_IMG * kh + _P * c + kw, k * _W + c] = 1.0
    return s


def _build_dsrc() -> np.ndarray:
    dinv = _dinv_grid()
    out = np.zeros((_NSTEPS, _R + 2, 4 * _W), dtype=np.float32)
    for i in range(_NSTEPS):
        for rl in range(_R + 2):
            r = i * _R + rl - 1
            if 0 <= r < _H:
                out[i, rl, :] = np.tile(dinv[r], 4)
    return out


def _build_lane_masks():
    ml = np.ones((1, 4 * _W), dtype=np.float32)
    mr = np.ones((1, 4 * _W), dtype=np.float32)
    for k in range(4):
        ml[0, k * _W] = 0.0
        mr[0, (k + 1) * _W - 1] = 0.0
    return ml, mr


_SM = _build_perm()
_DSRC = _build_dsrc()
_ML, _MR = _build_lane_masks()
_DDST = _dinv_grid().reshape(_NSTEPS, 1, _R * _W)


def _body(xpv_ref, xcv_ref, xnv_ref, sm_ref, wf_ref, wg_ref, b_ref, dv_ref,
          ml_ref, mr_ref, dd_ref, out_ref):
    i = pl.program_id(0)
    wct = jnp.dot(wg_ref[...], wf_ref[...], preferred_element_type=jnp.float32)

    xb = jnp.concatenate(
        [xpv_ref[2 * _R - 2:, :], xcv_ref[...], xnv_ref[:2, :]],
        axis=0)
    xh = jnp.concatenate(
        [jnp.concatenate([xb[2 * rl:2 * rl + 1, :],
                          xb[2 * rl + 1:2 * rl + 2, :]], axis=1)
         for rl in range(_R + 2)], axis=0)

    ut = jnp.dot(xh, sm_ref[...], preferred_element_type=jnp.float32)
    ut = ut * dv_ref[0]

    ct = ut[1:_R + 1]
    zc1 = jnp.zeros((_R, 1), jnp.float32)
    shl = jnp.concatenate([zc1, ct[:, :4 * _W - 1]], axis=1) * ml_ref[...]
    shr = jnp.concatenate([ct[:, 1:], zc1], axis=1) * mr_ref[...]
    usum = ut[0:_R] + ct + ut[2:_R + 2] + shl + shr

    pt = jnp.concatenate(
        [jnp.concatenate([usum[r:r + 1, k * _W:(k + 1) * _W]
                          for r in range(_R)], axis=1)
         for k in range(4)], axis=0)
    z = jnp.dot(wct, pt, preferred_element_type=jnp.float32)
    out_ref[...] = z * dd_ref[0] + b_ref[...]

    @pl.when(i == _NSTEPS - 1)
    def _():
        loc = ((_H - 2) % _R) * _W + (_W - 2)
        src = jnp.concatenate(
            [ut[_R:_R + 1, (k + 1) * _W - 1:(k + 1) * _W]
             for k in range(4)], axis=0)
        corr = np.float32(1.0 / np.sqrt(6.0)) * jnp.dot(
            wct, src, preferred_element_type=jnp.float32)
        out_ref[:, loc:loc + 1] = out_ref[:, loc:loc + 1] + corr


def kernel(x, Wconv, Wgcn, bgcn):
    xr = x.reshape(_IMG, _IMG)
    wf = Wconv.reshape(_C, 4).astype(jnp.float32)
    wg = Wgcn.astype(jnp.float32)
    b = bgcn.reshape(_C, 1).astype(jnp.float32)

    out = pl.pallas_call(
        _body,
        grid=(_NSTEPS,),
        in_specs=[
            pl.BlockSpec((2 * _R, _IMG), lambda i: (jnp.maximum(i - 1, 0), 0)),
            pl.BlockSpec((2 * _R, _IMG), lambda i: (i, 0)),
            pl.BlockSpec((2 * _R, _IMG),
                         lambda i: (jnp.minimum(i + 1, _NSTEPS - 1), 0)),
            pl.BlockSpec((2 * _IMG, 4 * _W), lambda i: (0, 0)),
            pl.BlockSpec((_C, 4), lambda i: (0, 0)),
            pl.BlockSpec((_C, _C), lambda i: (0, 0)),
            pl.BlockSpec((_C, 1), lambda i: (0, 0)),
            pl.BlockSpec((1, _R + 2, 4 * _W), lambda i: (i, 0, 0)),
            pl.BlockSpec((1, 4 * _W), lambda i: (0, 0)),
            pl.BlockSpec((1, 4 * _W), lambda i: (0, 0)),
            pl.BlockSpec((1, 1, _R * _W), lambda i: (i, 0, 0)),
        ],
        out_specs=pl.BlockSpec((_C, _R * _W), lambda i: (0, i)),
        out_shape=jax.ShapeDtypeStruct((_C, _N), jnp.float32),
    )(xr, xr, xr, jnp.asarray(_SM), wf, wg, b, jnp.asarray(_DSRC),
      jnp.asarray(_ML), jnp.asarray(_MR), jnp.asarray(_DDST))

    return out.T.reshape(1, _N, _C)


# speedup vs baseline: 1.1101x; 1.0695x over previous
import jax
import jax.numpy as jnp
import numpy as np
from jax.experimental import pallas as pl

_P = 2
_C = 96
_IMG = 512
_W = _IMG // _P
_H = _IMG // _P
_N = _H * _W
_R = 64
_NSTEPS = _H // _R


def _dinv_grid() -> np.ndarray:
    r = np.arange(_H)[:, None]
    c = np.arange(_W)[None, :]
    deg = (1.0 + (r > 0) + (r < _H - 1) + (c > 0) + (c < _W - 1)
           + ((r == _H - 2) & (c == _W - 2)))
    return (1.0 / np.sqrt(deg)).astype(np.float32)


def _build_perm() -> np.ndarray:
    s = np.zeros((_IMG, 2 * _W), dtype=np.float32)
    for kw in range(_P):
        for c in range(_W):
            s[_P * c + kw, kw * _W + c] = 1.0
    return s


def _build_dsrc() -> np.ndarray:
    dinv = _dinv_grid()
    out = np.zeros((_NSTEPS, _R + 2, 4 * _W), dtype=np.float32)
    for i in range(_NSTEPS):
        for rl in range(_R + 2):
            r = i * _R + rl - 1
            if 0 <= r < _H:
                out[i, rl, :] = np.tile(dinv[r], 4)
    return out


def _build_lane_masks():
    ml = np.ones((1, 4 * _W), dtype=np.float32)
    mr = np.ones((1, 4 * _W), dtype=np.float32)
    for k in range(4):
        ml[0, k * _W] = 0.0
        mr[0, (k + 1) * _W - 1] = 0.0
    return ml, mr


_SM = _build_perm()
_DSRC = _build_dsrc()
_ML, _MR = _build_lane_masks()
_DDST = _dinv_grid().reshape(_NSTEPS, 1, _R * _W)


def _body(xpv_ref, xcv_ref, xnv_ref, sm_ref, wf_ref, wg_ref, b_ref, dv_ref,
          ml_ref, mr_ref, dd_ref, out_ref):
    i = pl.program_id(0)
    wct = jnp.dot(wg_ref[...], wf_ref[...], preferred_element_type=jnp.float32)

    xb = jnp.concatenate(
        [xpv_ref[2 * _R - 2:, :], xcv_ref[...], xnv_ref[:2, :]],
        axis=0)
    xe = jnp.concatenate([xb[2 * rl:2 * rl + 1, :]
                          for rl in range(_R + 2)], axis=0)
    xo = jnp.concatenate([xb[2 * rl + 1:2 * rl + 2, :]
                          for rl in range(_R + 2)], axis=0)

    ut = jnp.concatenate(
        [jnp.dot(xe, sm_ref[...], preferred_element_type=jnp.float32),
         jnp.dot(xo, sm_ref[...], preferred_element_type=jnp.float32)],
        axis=1)
    ut = ut * dv_ref[0]

    ct = ut[1:_R + 1]
    zc1 = jnp.zeros((_R, 1), jnp.float32)
    shl = jnp.concatenate([zc1, ct[:, :4 * _W - 1]], axis=1) * ml_ref[...]
    shr = jnp.concatenate([ct[:, 1:], zc1], axis=1) * mr_ref[...]
    usum = ut[0:_R] + ct + ut[2:_R + 2] + shl + shr

    pt = jnp.concatenate(
        [jnp.concatenate([usum[r:r + 1, k * _W:(k + 1) * _W]
                          for r in range(_R)], axis=1)
         for k in range(4)], axis=0)
    z = jnp.dot(wct, pt, preferred_element_type=jnp.float32)
    out_ref[...] = z * dd_ref[0] + b_ref[...]

    @pl.when(i == _NSTEPS - 1)
    def _():
        loc = ((_H - 2) % _R) * _W + (_W - 2)
        src = jnp.concatenate(
            [ut[_R:_R + 1, (k + 1) * _W - 1:(k + 1) * _W]
             for k in range(4)], axis=0)
        corr = np.float32(1.0 / np.sqrt(6.0)) * jnp.dot(
            wct, src, preferred_element_type=jnp.float32)
        out_ref[:, loc:loc + 1] = out_ref[:, loc:loc + 1] + corr


def kernel(x, Wconv, Wgcn, bgcn):
    xr = x.reshape(_IMG, _IMG)
    wf = Wconv.reshape(_C, 4).astype(jnp.float32)
    wg = Wgcn.astype(jnp.float32)
    b = bgcn.reshape(_C, 1).astype(jnp.float32)

    out = pl.pallas_call(
        _body,
        grid=(_NSTEPS,),
        in_specs=[
            pl.BlockSpec((2 * _R, _IMG), lambda i: (jnp.maximum(i - 1, 0), 0)),
            pl.BlockSpec((2 * _R, _IMG), lambda i: (i, 0)),
            pl.BlockSpec((2 * _R, _IMG),
                         lambda i: (jnp.minimum(i + 1, _NSTEPS - 1), 0)),
            pl.BlockSpec((_IMG, 2 * _W), lambda i: (0, 0)),
            pl.BlockSpec((_C, 4), lambda i: (0, 0)),
            pl.BlockSpec((_C, _C), lambda i: (0, 0)),
            pl.BlockSpec((_C, 1), lambda i: (0, 0)),
            pl.BlockSpec((1, _R + 2, 4 * _W), lambda i: (i, 0, 0)),
            pl.BlockSpec((1, 4 * _W), lambda i: (0, 0)),
            pl.BlockSpec((1, 4 * _W), lambda i: (0, 0)),
            pl.BlockSpec((1, 1, _R * _W), lambda i: (i, 0, 0)),
        ],
        out_specs=pl.BlockSpec((_C, _R * _W), lambda i: (0, i)),
        out_shape=jax.ShapeDtypeStruct((_C, _N), jnp.float32),
    )(xr, xr, xr, jnp.asarray(_SM), wf, wg, b, jnp.asarray(_DSRC),
      jnp.asarray(_ML), jnp.asarray(_MR), jnp.asarray(_DDST))

    return out.T.reshape(1, _N, _C)
